# baseline (speedup 1.0000x reference)
import functools

import jax
import jax.numpy as jnp
from jax import lax
from jax.experimental import pallas as pl
from jax.experimental.pallas import tpu as pltpu
from jax.experimental.pallas import tpu_sc as plsc

N_NODES = 100000
N_EDGES = 3200000
D_NODE = 128
D_EDGE = 16
D_GLOBAL = 16
D_OUT = 128

NUM_CORES = 2
NUM_SUBCORES = 16
NUM_TILES = NUM_CORES * NUM_SUBCORES

CHUNK = 256
SCAT = 128
SUB = CHUNK // SCAT
N_CHUNKS = N_EDGES // CHUNK
ROUNDS = -(-N_CHUNKS // NUM_TILES)
SUPER = (ROUNDS + 1) // 2

ROWS_PER_SUBCORE = N_NODES // NUM_SUBCORES

BLK = 2000
N_BLKS = N_NODES // BLK


def _sc_segment_sum(edges_t, recv3):
    mesh = plsc.VectorSubcoreMesh(core_axis_name="c", subcore_axis_name="s")

    @functools.partial(
        pl.kernel,
        out_type=jax.ShapeDtypeStruct((NUM_CORES, N_NODES, D_EDGE), jnp.float32),
        mesh=mesh,
        compiler_params=pltpu.CompilerParams(
            use_tc_tiling_on_sc=False, needs_layout_passes=False
        ),
        scratch_types=[
            pltpu.VMEM_SHARED((N_NODES, D_EDGE), jnp.float32),
            pltpu.VMEM((2, D_EDGE, CHUNK), jnp.float32),
            pltpu.VMEM((2, CHUNK, D_EDGE), jnp.float32),
            pltpu.VMEM((4, SUB, SCAT), jnp.int32),
            pltpu.SemaphoreType.DMA,
            pltpu.SemaphoreType.DMA,
            pltpu.SemaphoreType.DMA,
            pltpu.SemaphoreType.DMA,
        ],
    )
    def sc_kernel(et_hbm, i_hbm, out_hbm, acc, etbuf, ebuf, ibuf,
                  lsem0, lsem1, ssem0, ssem1):
        cid = lax.axis_index("c")
        sid = lax.axis_index("s")
        wid = sid * NUM_CORES + cid
        lsem = (lsem0, lsem1)
        ssem = (ssem0, ssem1)

        zstage = ebuf.at[0]

        @pl.loop(0, CHUNK)
        def _(i):
            zstage[i, :] = jnp.zeros((D_EDGE,), jnp.float32)

        @pl.loop(0, ROWS_PER_SUBCORE // CHUNK)
        def _(k):
            pltpu.sync_copy(
                zstage, acc.at[pl.ds(sid * ROWS_PER_SUBCORE + k * CHUNK, CHUNK)]
            )

        _tail_base = sid * ROWS_PER_SUBCORE + (ROWS_PER_SUBCORE // CHUNK) * CHUNK
        _tail = ROWS_PER_SUBCORE % CHUNK
        pltpu.sync_copy(zstage.at[pl.ds(0, _tail)], acc.at[pl.ds(_tail_base, _tail)])

        plsc.subcore_barrier()

        lane = lax.iota(jnp.int32, 16)
        perms = [jnp.bitwise_and(lane + k, 15) for k in range(16)]

        def start_load(slot, rd):
            c = wid + NUM_TILES * rd

            @pl.when(c < N_CHUNKS)
            def _():
                pltpu.async_copy(
                    et_hbm.at[:, pl.ds(c * CHUNK, CHUNK)], etbuf.at[slot],
                    lsem[slot])
                pltpu.async_copy(i_hbm.at[c], ibuf.at[rd % 4], lsem[slot])

        def wait_load(slot):
            pltpu.make_async_copy(
                et_hbm.at[:, pl.ds(0, CHUNK)], etbuf.at[slot], lsem[slot]).wait()
            pltpu.make_async_copy(
                i_hbm.at[0], ibuf.at[0], lsem[slot]).wait()

        def wait_scatter(slot):
            pltpu.make_async_copy(
                out_hbm.at[0, pl.ds(0, CHUNK)], ebuf.at[slot], ssem[slot]).wait()

        start_load(0, 0)
        start_load(1, 1)

        @pl.loop(0, SUPER)
        def _(r):
            for slot in range(2):
                rd = 2 * r + slot
                c = wid + NUM_TILES * rd

                @pl.when(c < N_CHUNKS)
                def _():
                    wait_load(slot)

                    @pl.when(rd >= 2)
                    def _():
                        wait_scatter(slot)

                    @pl.loop(0, CHUNK, step=16)
                    def _(e0):
                        e0vec = jnp.full((16,), e0, jnp.int32)
                        diag = [e0vec + perms[k] for k in range(16)]
                        vals = [
                            plsc.load_gather(etbuf.at[slot], [lane, diag[k]])
                            for k in range(16)
                        ]
                        for k in range(16):
                            plsc.store_scatter(
                                ebuf.at[slot], [diag[k], lane], vals[k])

                    for j in range(SUB):
                        pltpu.async_copy(
                            ebuf.at[slot, pl.ds(j * SCAT, SCAT)],
                            acc.at[ibuf.at[rd % 4, j]],
                            ssem[slot],
                            add=True,
                        )
                    start_load(slot, rd + 2)

        wait_scatter(0)
        wait_scatter(1)

        plsc.subcore_barrier()

        pltpu.sync_copy(
            acc.at[pl.ds(sid * ROWS_PER_SUBCORE, ROWS_PER_SUBCORE)],
            out_hbm.at[cid, pl.ds(sid * ROWS_PER_SUBCORE, ROWS_PER_SUBCORE)],
        )

    return sc_kernel(edges_t, recv3)


def _tc_dense_kernel(n_ref, p_ref, g_ref, w_ref, b_ref, o_ref):
    x = n_ref[...]
    ps = p_ref[0] + p_ref[1]
    wn = w_ref[0:D_NODE, :]
    we = w_ref[D_NODE:D_NODE + D_EDGE, :]
    wg = w_ref[D_NODE + D_EDGE:, :]
    y = jnp.dot(x, wn, precision=lax.Precision.HIGHEST)
    y = y + jnp.dot(ps, we, precision=lax.Precision.HIGHEST)
    y = y + jnp.dot(g_ref[...], wg, precision=lax.Precision.HIGHEST)
    y = y + b_ref[...]
    o_ref[...] = jnp.maximum(y, 0.0)


def _tc_dense(nodes, partials, globals_, W, b2):
    return pl.pallas_call(
        _tc_dense_kernel,
        grid=(N_BLKS,),
        in_specs=[
            pl.BlockSpec((BLK, D_NODE), lambda i: (i, 0)),
            pl.BlockSpec((NUM_CORES, BLK, D_EDGE), lambda i: (0, i, 0)),
            pl.BlockSpec((1, D_GLOBAL), lambda i: (0, 0)),
            pl.BlockSpec((D_NODE + D_EDGE + D_GLOBAL, D_OUT), lambda i: (0, 0)),
            pl.BlockSpec((1, D_OUT), lambda i: (0, 0)),
        ],
        out_specs=pl.BlockSpec((BLK, D_OUT), lambda i: (i, 0)),
        out_shape=jax.ShapeDtypeStruct((N_NODES, D_OUT), jnp.float32),
    )(nodes, partials, globals_, W, b2)


def kernel(nodes, edges, receivers, senders, globals_, W, b):
    del senders
    recv3 = receivers.astype(jnp.int32).reshape(N_CHUNKS, SUB, SCAT)
    partials = _sc_segment_sum(edges.T, recv3)
    return _tc_dense(nodes, partials, globals_, W, b.reshape(1, D_OUT))


# speedup vs baseline: 1.2672x; 1.2672x over previous
import functools

import jax
import jax.numpy as jnp
from jax import lax
from jax.experimental import pallas as pl
from jax.experimental.pallas import tpu as pltpu
from jax.experimental.pallas import tpu_sc as plsc

N_NODES = 100000
N_EDGES = 3200000
D_NODE = 128
D_EDGE = 16
D_GLOBAL = 16
D_OUT = 128

NUM_CORES = 2
NUM_SUBCORES = 16
NUM_TILES = NUM_CORES * NUM_SUBCORES

LANES = 128
TCOLS = 2
CHUNK = TCOLS * LANES
SCAT = 128
SUB = CHUNK // SCAT
N_CHUNKS = N_EDGES // CHUNK
N_TILE_COLS = N_EDGES // LANES
ROUNDS = -(-N_CHUNKS // NUM_TILES)
SUPER = (ROUNDS + 1) // 2

ROWS_PER_SUBCORE = N_NODES // NUM_SUBCORES

BLK = 2000
N_BLKS = N_NODES // BLK


def _sc_segment_sum(e4, idx2):
    mesh = plsc.VectorSubcoreMesh(core_axis_name="c", subcore_axis_name="s")

    @functools.partial(
        pl.kernel,
        out_type=jax.ShapeDtypeStruct((NUM_CORES, N_NODES, D_EDGE), jnp.float32),
        mesh=mesh,
        compiler_params=pltpu.CompilerParams(
            use_tc_tiling_on_sc=False, needs_layout_passes=False
        ),
        scratch_types=[
            pltpu.VMEM_SHARED((N_NODES, D_EDGE), jnp.float32),
            pltpu.VMEM((2, 2, TCOLS, 8, LANES), jnp.float32),
            pltpu.VMEM((2, CHUNK, D_EDGE), jnp.float32),
            pltpu.VMEM((4, SUB, SCAT), jnp.int32),
            pltpu.SemaphoreType.DMA,
            pltpu.SemaphoreType.DMA,
            pltpu.SemaphoreType.DMA,
            pltpu.SemaphoreType.DMA,
        ],
    )
    def sc_kernel(e_hbm, i_hbm, out_hbm, acc, etbuf, ebuf, ibuf,
                  lsem0, lsem1, ssem0, ssem1):
        cid = lax.axis_index("c")
        sid = lax.axis_index("s")
        wid = sid * NUM_CORES + cid
        lsem = (lsem0, lsem1)
        ssem = (ssem0, ssem1)

        zstage = ebuf.at[0]

        @pl.loop(0, CHUNK)
        def _(i):
            zstage[i, :] = jnp.zeros((D_EDGE,), jnp.float32)

        @pl.loop(0, ROWS_PER_SUBCORE // CHUNK)
        def _(k):
            pltpu.sync_copy(
                zstage, acc.at[pl.ds(sid * ROWS_PER_SUBCORE + k * CHUNK, CHUNK)]
            )

        _tail_base = sid * ROWS_PER_SUBCORE + (ROWS_PER_SUBCORE // CHUNK) * CHUNK
        _tail = ROWS_PER_SUBCORE % CHUNK
        pltpu.sync_copy(zstage.at[pl.ds(0, _tail)], acc.at[pl.ds(_tail_base, _tail)])

        plsc.subcore_barrier()

        lane = lax.iota(jnp.int32, 16)
        perms = [jnp.bitwise_and(lane + k, 15) for k in range(16)]
        fhi = lax.shift_right_logical(lane, 3)
        flo = jnp.bitwise_and(lane, 7)

        def start_load(slot, rd):
            c = wid + NUM_TILES * rd

            @pl.when(c < N_CHUNKS)
            def _():
                pltpu.async_copy(
                    e_hbm.at[:, pl.ds(c * TCOLS, TCOLS)], etbuf.at[slot],
                    lsem[slot])
                pltpu.async_copy(
                    i_hbm.at[pl.ds(c * SUB, SUB)], ibuf.at[rd % 4], lsem[slot])

        def wait_load(slot):
            pltpu.make_async_copy(
                e_hbm.at[:, pl.ds(0, TCOLS)], etbuf.at[slot], lsem[slot]).wait()
            pltpu.make_async_copy(
                i_hbm.at[pl.ds(0, SUB)], ibuf.at[0], lsem[slot]).wait()

        def wait_scatter(slot):
            pltpu.make_async_copy(
                out_hbm.at[0, pl.ds(0, CHUNK)], ebuf.at[slot], ssem[slot]).wait()

        start_load(0, 0)
        start_load(1, 1)

        @pl.loop(0, SUPER)
        def _(r):
            for slot in range(2):
                rd = 2 * r + slot
                c = wid + NUM_TILES * rd

                @pl.when(c < N_CHUNKS)
                def _():
                    wait_load(slot)

                    @pl.when(rd >= 2)
                    def _():
                        wait_scatter(slot)

                    @pl.loop(0, LANES, step=16)
                    def _(l0):
                        for jj in range(TCOLS):
                            base = jnp.full((16,), jj * LANES + l0, jnp.int32)
                            l0v = jnp.full((16,), l0, jnp.int32)
                            diag = [l0v + perms[k] for k in range(16)]
                            rowd = [base + perms[k] for k in range(16)]
                            vals = [
                                plsc.load_gather(
                                    etbuf.at[slot],
                                    [fhi, jnp.full((16,), jj, jnp.int32),
                                     flo, diag[k]])
                                for k in range(16)
                            ]
                            for k in range(16):
                                plsc.store_scatter(
                                    ebuf.at[slot], [rowd[k], lane], vals[k])

                    for j in range(SUB):
                        pltpu.async_copy(
                            ebuf.at[slot, pl.ds(j * SCAT, SCAT)],
                            acc.at[ibuf.at[rd % 4, j]],
                            ssem[slot],
                            add=True,
                        )
                    start_load(slot, rd + 2)

        wait_scatter(0)
        wait_scatter(1)

        plsc.subcore_barrier()

        pltpu.sync_copy(
            acc.at[pl.ds(sid * ROWS_PER_SUBCORE, ROWS_PER_SUBCORE)],
            out_hbm.at[cid, pl.ds(sid * ROWS_PER_SUBCORE, ROWS_PER_SUBCORE)],
        )

    return sc_kernel(e4, idx2)


def _tc_dense_kernel(n_ref, p_ref, g_ref, w_ref, b_ref, o_ref):
    x = n_ref[...]
    ps = p_ref[0] + p_ref[1]
    wn = w_ref[0:D_NODE, :]
    we = w_ref[D_NODE:D_NODE + D_EDGE, :]
    wg = w_ref[D_NODE + D_EDGE:, :]
    y = jnp.dot(x, wn, precision=lax.Precision.HIGHEST)
    y = y + jnp.dot(ps, we, precision=lax.Precision.HIGHEST)
    y = y + jnp.dot(g_ref[...], wg, precision=lax.Precision.HIGHEST)
    y = y + b_ref[...]
    o_ref[...] = jnp.maximum(y, 0.0)


def _tc_dense(nodes, partials, globals_, W, b2):
    return pl.pallas_call(
        _tc_dense_kernel,
        grid=(N_BLKS,),
        in_specs=[
            pl.BlockSpec((BLK, D_NODE), lambda i: (i, 0)),
            pl.BlockSpec((NUM_CORES, BLK, D_EDGE), lambda i: (0, i, 0)),
            pl.BlockSpec((1, D_GLOBAL), lambda i: (0, 0)),
            pl.BlockSpec((D_NODE + D_EDGE + D_GLOBAL, D_OUT), lambda i: (0, 0)),
            pl.BlockSpec((1, D_OUT), lambda i: (0, 0)),
        ],
        out_specs=pl.BlockSpec((BLK, D_OUT), lambda i: (i, 0)),
        out_shape=jax.ShapeDtypeStruct((N_NODES, D_OUT), jnp.float32),
    )(nodes, partials, globals_, W, b2)


def kernel(nodes, edges, receivers, senders, globals_, W, b):
    del senders
    idx2 = receivers.astype(jnp.int32).reshape(N_TILE_COLS, LANES)
    e4 = edges.T.reshape(NUM_CORES, 8, N_TILE_COLS, LANES).transpose(0, 2, 1, 3)
    partials = _sc_segment_sum(e4, idx2)
    return _tc_dense(nodes, partials, globals_, W, b.reshape(1, D_OUT))


# speedup vs baseline: 1.4348x; 1.1322x over previous
import functools

import jax
import jax.numpy as jnp
from jax import lax
from jax.experimental import pallas as pl
from jax.experimental.pallas import tpu as pltpu
from jax.experimental.pallas import tpu_sc as plsc

N_NODES = 100000
N_EDGES = 3200000
D_NODE = 128
D_EDGE = 16
D_GLOBAL = 16
D_OUT = 128

NUM_CORES = 2
NUM_SUBCORES = 16
NUM_TILES = NUM_CORES * NUM_SUBCORES

LANES = 128
TCOLS = 2
CHUNK = TCOLS * LANES
SCAT = 128
SUB = CHUNK // SCAT
N_CHUNKS = N_EDGES // CHUNK
N_TILE_COLS = N_EDGES // LANES
ROUNDS = -(-N_CHUNKS // NUM_TILES)
SUPER = (ROUNDS + 1) // 2

ROWS_PER_SUBCORE = N_NODES // NUM_SUBCORES

BLK = 2000
N_BLKS = N_NODES // BLK


def _sc_segment_sum(e4, idx2):
    mesh = plsc.VectorSubcoreMesh(core_axis_name="c", subcore_axis_name="s")

    @functools.partial(
        pl.kernel,
        out_type=jax.ShapeDtypeStruct((NUM_CORES, N_NODES, D_EDGE), jnp.float32),
        mesh=mesh,
        compiler_params=pltpu.CompilerParams(
            use_tc_tiling_on_sc=False, needs_layout_passes=False
        ),
        scratch_types=[
            pltpu.VMEM_SHARED((N_NODES, D_EDGE), jnp.float32),
            pltpu.VMEM((2, 2, TCOLS, 8, LANES), jnp.float32),
            pltpu.VMEM((2, TCOLS, LANES, D_EDGE), jnp.float32),
            pltpu.VMEM((4, SUB, SCAT), jnp.int32),
            pltpu.SemaphoreType.DMA,
            pltpu.SemaphoreType.DMA,
            pltpu.SemaphoreType.DMA,
            pltpu.SemaphoreType.DMA,
        ],
    )
    def sc_kernel(e_hbm, i_hbm, out_hbm, acc, etbuf, ebuf, ibuf,
                  lsem0, lsem1, ssem0, ssem1):
        cid = lax.axis_index("c")
        sid = lax.axis_index("s")
        wid = sid * NUM_CORES + cid
        lsem = (lsem0, lsem1)
        ssem = (ssem0, ssem1)

        zstage = ebuf.at[0, 0]

        @pl.loop(0, LANES)
        def _(i):
            zstage[i, :] = jnp.zeros((D_EDGE,), jnp.float32)

        @pl.loop(0, ROWS_PER_SUBCORE // LANES)
        def _(k):
            pltpu.sync_copy(
                zstage, acc.at[pl.ds(sid * ROWS_PER_SUBCORE + k * LANES, LANES)]
            )

        _tail_base = sid * ROWS_PER_SUBCORE + (ROWS_PER_SUBCORE // LANES) * LANES
        _tail = ROWS_PER_SUBCORE % LANES
        pltpu.sync_copy(zstage.at[pl.ds(0, _tail)], acc.at[pl.ds(_tail_base, _tail)])

        plsc.subcore_barrier()

        lane = lax.iota(jnp.int32, 16)
        perms = [jnp.bitwise_and(lane + k, 15) for k in range(16)]
        fhi = lax.shift_right_logical(lane, 3)
        flo = jnp.bitwise_and(lane, 7)

        def start_load(slot, rd):
            c = wid + NUM_TILES * rd

            @pl.when(c < N_CHUNKS)
            def _():
                pltpu.async_copy(
                    e_hbm.at[:, pl.ds(c * TCOLS, TCOLS)], etbuf.at[slot],
                    lsem[slot])
                pltpu.async_copy(
                    i_hbm.at[pl.ds(c * SUB, SUB)], ibuf.at[rd % 4], lsem[slot])

        def wait_load(slot):
            pltpu.make_async_copy(
                e_hbm.at[:, pl.ds(0, TCOLS)], etbuf.at[slot], lsem[slot]).wait()
            pltpu.make_async_copy(
                i_hbm.at[pl.ds(0, SUB)], ibuf.at[0], lsem[slot]).wait()

        def wait_scatter(slot):
            for j in range(SUB):
                pltpu.make_async_copy(
                    out_hbm.at[0, pl.ds(0, SCAT)], ebuf.at[slot, j],
                    ssem[slot]).wait()

        start_load(0, 0)
        start_load(1, 1)

        @pl.loop(0, SUPER)
        def _(r):
            for slot in range(2):
                rd = 2 * r + slot
                c = wid + NUM_TILES * rd

                @pl.when(c < N_CHUNKS)
                def _():
                    wait_load(slot)

                    @pl.when(rd >= 2)
                    def _():
                        wait_scatter(slot)

                    for jj in range(TCOLS):
                        jjv = jnp.full((16,), jj, jnp.int32)

                        @pl.loop(0, LANES, step=16)
                        def _(l0):
                            diag = [
                                jnp.full((16,), l0, jnp.int32) + perms[k]
                                for k in range(16)
                            ]
                            vals = [
                                plsc.load_gather(
                                    etbuf.at[slot], [fhi, jjv, flo, diag[k]])
                                for k in range(16)
                            ]
                            for k in range(16):
                                plsc.store_scatter(
                                    ebuf.at[slot],
                                    [jjv, diag[k], lane], vals[k])

                        pltpu.async_copy(
                            ebuf.at[slot, jj],
                            acc.at[ibuf.at[rd % 4, jj]],
                            ssem[slot],
                            add=True,
                        )
                    start_load(slot, rd + 2)

        wait_scatter(0)
        wait_scatter(1)

        plsc.subcore_barrier()

        pltpu.sync_copy(
            acc.at[pl.ds(sid * ROWS_PER_SUBCORE, ROWS_PER_SUBCORE)],
            out_hbm.at[cid, pl.ds(sid * ROWS_PER_SUBCORE, ROWS_PER_SUBCORE)],
        )

    return sc_kernel(e4, idx2)


def _tc_dense_kernel(n_ref, p_ref, g_ref, w_ref, b_ref, o_ref):
    x = n_ref[...]
    ps = p_ref[0] + p_ref[1]
    wn = w_ref[0:D_NODE, :]
    we = w_ref[D_NODE:D_NODE + D_EDGE, :]
    wg = w_ref[D_NODE + D_EDGE:, :]
    y = jnp.dot(x, wn, precision=lax.Precision.DEFAULT)
    y = y + jnp.dot(ps, we, precision=lax.Precision.DEFAULT)
    y = y + jnp.dot(g_ref[...], wg, precision=lax.Precision.DEFAULT)
    y = y + b_ref[...]
    o_ref[...] = jnp.maximum(y, 0.0)


def _tc_dense(nodes, partials, globals_, W, b2):
    return pl.pallas_call(
        _tc_dense_kernel,
        grid=(N_BLKS,),
        in_specs=[
            pl.BlockSpec((BLK, D_NODE), lambda i: (i, 0)),
            pl.BlockSpec((NUM_CORES, BLK, D_EDGE), lambda i: (0, i, 0)),
            pl.BlockSpec((1, D_GLOBAL), lambda i: (0, 0)),
            pl.BlockSpec((D_NODE + D_EDGE + D_GLOBAL, D_OUT), lambda i: (0, 0)),
            pl.BlockSpec((1, D_OUT), lambda i: (0, 0)),
        ],
        out_specs=pl.BlockSpec((BLK, D_OUT), lambda i: (i, 0)),
        out_shape=jax.ShapeDtypeStruct((N_NODES, D_OUT), jnp.float32),
    )(nodes, partials, globals_, W, b2)


def kernel(nodes, edges, receivers, senders, globals_, W, b):
    del senders
    idx2 = receivers.astype(jnp.int32).reshape(N_TILE_COLS, LANES)
    e4 = edges.T.reshape(NUM_CORES, 8, N_TILE_COLS, LANES).transpose(0, 2, 1, 3)
    partials = _sc_segment_sum(e4, idx2)
    return _tc_dense(nodes, partials, globals_, W, b.reshape(1, D_OUT))


# speedup vs baseline: 1.5238x; 1.0621x over previous
import functools

import jax
import jax.numpy as jnp
from jax import lax
from jax.experimental import pallas as pl
from jax.experimental.pallas import tpu as pltpu
from jax.experimental.pallas import tpu_sc as plsc

N_NODES = 100000
N_EDGES = 3200000
D_NODE = 128
D_EDGE = 16
D_GLOBAL = 16
D_OUT = 128

NUM_CORES = 2
NUM_SUBCORES = 16
NUM_TILES = NUM_CORES * NUM_SUBCORES

LANES = 128
TCOLS = 2
CHUNK = TCOLS * LANES
SCAT = 128
SUB = CHUNK // SCAT
N_CHUNKS = N_EDGES // CHUNK
N_TILE_COLS = N_EDGES // LANES
ROUNDS = -(-N_CHUNKS // NUM_TILES)
SUPER = (ROUNDS + 1) // 2

ROWS_PER_SUBCORE = N_NODES // NUM_SUBCORES

BLK = 2048
N_BLKS = -(-N_NODES // BLK)


def _sc_segment_sum(e4, idx2):
    mesh = plsc.VectorSubcoreMesh(core_axis_name="c", subcore_axis_name="s")

    @functools.partial(
        pl.kernel,
        out_type=jax.ShapeDtypeStruct((NUM_CORES * N_NODES * D_EDGE,), jnp.float32),
        mesh=mesh,
        compiler_params=pltpu.CompilerParams(
            use_tc_tiling_on_sc=False, needs_layout_passes=False
        ),
        scratch_types=[
            pltpu.VMEM_SHARED((N_NODES, D_EDGE), jnp.float32),
            pltpu.VMEM((2, 2, TCOLS, 8, LANES), jnp.float32),
            pltpu.VMEM((2, TCOLS, LANES, D_EDGE), jnp.float32),
            pltpu.VMEM((4, SUB, SCAT), jnp.int32),
            pltpu.VMEM((2, SCAT * D_EDGE), jnp.float32),
            pltpu.SemaphoreType.DMA,
            pltpu.SemaphoreType.DMA,
            pltpu.SemaphoreType.DMA,
            pltpu.SemaphoreType.DMA,
        ],
    )
    def sc_kernel(e_hbm, i_hbm, out_hbm, acc, etbuf, ebuf, ibuf, obuf,
                  lsem0, lsem1, ssem0, ssem1):
        cid = lax.axis_index("c")
        sid = lax.axis_index("s")
        wid = sid * NUM_CORES + cid
        lsem = (lsem0, lsem1)
        ssem = (ssem0, ssem1)

        zstage = ebuf.at[0, 0]

        @pl.loop(0, LANES)
        def _(i):
            zstage[i, :] = jnp.zeros((D_EDGE,), jnp.float32)

        @pl.loop(0, ROWS_PER_SUBCORE // LANES)
        def _(k):
            pltpu.sync_copy(
                zstage, acc.at[pl.ds(sid * ROWS_PER_SUBCORE + k * LANES, LANES)]
            )

        _tail_base = sid * ROWS_PER_SUBCORE + (ROWS_PER_SUBCORE // LANES) * LANES
        _tail = ROWS_PER_SUBCORE % LANES
        pltpu.sync_copy(zstage.at[pl.ds(0, _tail)], acc.at[pl.ds(_tail_base, _tail)])

        plsc.subcore_barrier()

        lane = lax.iota(jnp.int32, 16)
        perms = [jnp.bitwise_and(lane + k, 15) for k in range(16)]
        fhi = lax.shift_right_logical(lane, 3)
        flo = jnp.bitwise_and(lane, 7)

        def start_load(slot, rd):
            c = wid + NUM_TILES * rd

            @pl.when(c < N_CHUNKS)
            def _():
                pltpu.async_copy(
                    e_hbm.at[:, pl.ds(c * TCOLS, TCOLS)], etbuf.at[slot],
                    lsem[slot])
                pltpu.async_copy(
                    i_hbm.at[pl.ds(c * SUB, SUB)], ibuf.at[rd % 4], lsem[slot])

        def wait_load(slot):
            pltpu.make_async_copy(
                e_hbm.at[:, pl.ds(0, TCOLS)], etbuf.at[slot], lsem[slot]).wait()
            pltpu.make_async_copy(
                i_hbm.at[pl.ds(0, SUB)], ibuf.at[0], lsem[slot]).wait()

        def wait_scatter(slot):
            for j in range(SUB):
                pltpu.make_async_copy(
                    out_hbm.at[pl.ds(0, SCAT * D_EDGE)], obuf.at[0],
                    ssem[slot]).wait()

        start_load(0, 0)
        start_load(1, 1)

        @pl.loop(0, SUPER)
        def _(r):
            for slot in range(2):
                rd = 2 * r + slot
                c = wid + NUM_TILES * rd

                @pl.when(c < N_CHUNKS)
                def _():
                    wait_load(slot)

                    @pl.when(rd >= 2)
                    def _():
                        wait_scatter(slot)

                    for jj in range(TCOLS):
                        jjv = jnp.full((16,), jj, jnp.int32)

                        @pl.loop(0, LANES, step=16)
                        def _(l0):
                            diag = [
                                jnp.full((16,), l0, jnp.int32) + perms[k]
                                for k in range(16)
                            ]
                            vals = [
                                plsc.load_gather(
                                    etbuf.at[slot], [fhi, jjv, flo, diag[k]])
                                for k in range(16)
                            ]
                            for k in range(16):
                                plsc.store_scatter(
                                    ebuf.at[slot],
                                    [jjv, diag[k], lane], vals[k])

                        pltpu.async_copy(
                            ebuf.at[slot, jj],
                            acc.at[ibuf.at[rd % 4, jj]],
                            ssem[slot],
                            add=True,
                        )
                    start_load(slot, rd + 2)

        wait_scatter(0)
        wait_scatter(1)

        plsc.subcore_barrier()

        OFULL = ROWS_PER_SUBCORE // SCAT
        OTAIL = ROWS_PER_SUBCORE % SCAT
        obase = (cid * N_NODES + sid * ROWS_PER_SUBCORE) * D_EDGE

        def relabel(oslot, nrows):
            t2 = ebuf.at[oslot, 0]

            @pl.loop(0, nrows)
            def _(i):
                obuf[oslot, pl.ds(i * D_EDGE, D_EDGE)] = t2[i, :]

        def owait(oslot, nrows):
            pltpu.make_async_copy(
                obuf.at[oslot, pl.ds(0, nrows * D_EDGE)],
                out_hbm.at[pl.ds(0, nrows * D_EDGE)], ssem[oslot]).wait()

        @pl.loop(0, OFULL // 2)
        def _(r):
            for oslot in range(2):
                k2 = 2 * r + oslot
                pltpu.sync_copy(
                    acc.at[pl.ds(sid * ROWS_PER_SUBCORE + k2 * SCAT, SCAT)],
                    ebuf.at[oslot, 0])

                @pl.when(r >= 1)
                def _():
                    owait(oslot, SCAT)

                relabel(oslot, SCAT)
                pltpu.async_copy(
                    obuf.at[oslot],
                    out_hbm.at[pl.ds(obase + k2 * SCAT * D_EDGE,
                                     SCAT * D_EDGE)],
                    ssem[oslot])

        pltpu.sync_copy(
            acc.at[pl.ds(sid * ROWS_PER_SUBCORE + OFULL * SCAT, OTAIL)],
            ebuf.at[0, 0, pl.ds(0, OTAIL)])
        owait(0, SCAT)
        relabel(0, OTAIL)
        pltpu.async_copy(
            obuf.at[0, pl.ds(0, OTAIL * D_EDGE)],
            out_hbm.at[pl.ds(obase + OFULL * SCAT * D_EDGE, OTAIL * D_EDGE)],
            ssem0)
        owait(1, SCAT)
        owait(0, OTAIL)

    return sc_kernel(e4, idx2)


PROWS = BLK * D_EDGE // 128


def _tc_dense_kernel(n_ref, p_ref, g_ref, w_ref, b_ref, o_ref):
    x = n_ref[...]
    pv = p_ref[0] + p_ref[1]
    wn = w_ref[0:D_NODE, :]
    we = w_ref[D_NODE:D_NODE + D_EDGE, :]
    wg = w_ref[D_NODE + D_EDGE:, :]
    y = jnp.dot(x, wn, precision=lax.Precision.DEFAULT)
    zs = [
        jnp.dot(pv[:, 16 * j:16 * (j + 1)], we,
                precision=lax.Precision.DEFAULT)
        for j in range(8)
    ]
    y = y + jnp.stack(zs, axis=1).reshape(BLK, D_OUT)
    y = y + jnp.dot(g_ref[...], wg, precision=lax.Precision.DEFAULT)
    y = y + b_ref[...]
    o_ref[...] = jnp.maximum(y, 0.0)


def _tc_dense(nodes, partials, globals_, W, b2):
    return pl.pallas_call(
        _tc_dense_kernel,
        grid=(N_BLKS,),
        in_specs=[
            pl.BlockSpec((BLK, D_NODE), lambda i: (i, 0)),
            pl.BlockSpec((NUM_CORES, PROWS, 128), lambda i: (0, i, 0)),
            pl.BlockSpec((1, D_GLOBAL), lambda i: (0, 0)),
            pl.BlockSpec((D_NODE + D_EDGE + D_GLOBAL, D_OUT), lambda i: (0, 0)),
            pl.BlockSpec((1, D_OUT), lambda i: (0, 0)),
        ],
        out_specs=pl.BlockSpec((BLK, D_OUT), lambda i: (i, 0)),
        out_shape=jax.ShapeDtypeStruct((N_NODES, D_OUT), jnp.float32),
    )(nodes, partials, globals_, W, b2)


def kernel(nodes, edges, receivers, senders, globals_, W, b):
    del senders
    idx2 = receivers.astype(jnp.int32).reshape(N_TILE_COLS, LANES)
    e4 = edges.T.reshape(NUM_CORES, 8, N_TILE_COLS, LANES).transpose(0, 2, 1, 3)
    partials1d = _sc_segment_sum(e4, idx2)
    pv = partials1d.reshape(NUM_CORES, N_NODES * D_EDGE // 128, 128)
    return _tc_dense(nodes, pv, globals_, W, b.reshape(1, D_OUT))


# speedup vs baseline: 1.5294x; 1.0036x over previous
import functools

import jax
import jax.numpy as jnp
from jax import lax
from jax.experimental import pallas as pl
from jax.experimental.pallas import tpu as pltpu
from jax.experimental.pallas import tpu_sc as plsc

N_NODES = 100000
N_EDGES = 3200000
D_NODE = 128
D_EDGE = 16
D_GLOBAL = 16
D_OUT = 128

NUM_CORES = 2
NUM_SUBCORES = 16
NUM_TILES = NUM_CORES * NUM_SUBCORES

LANES = 128
TCOLS = 2
CHUNK = TCOLS * LANES
SCAT = 128
SUB = CHUNK // SCAT
N_CHUNKS = N_EDGES // CHUNK
N_TILE_COLS = N_EDGES // LANES
ROUNDS = -(-N_CHUNKS // NUM_TILES)
SUPER = (ROUNDS + 1) // 2

ROWS_PER_SUBCORE = N_NODES // NUM_SUBCORES

BLK = 2048
N_BLKS = -(-N_NODES // BLK)


def _sc_segment_sum(e4, idx2):
    mesh = plsc.VectorSubcoreMesh(core_axis_name="c", subcore_axis_name="s")

    @functools.partial(
        pl.kernel,
        out_type=jax.ShapeDtypeStruct((NUM_CORES * N_NODES * D_EDGE,), jnp.float32),
        mesh=mesh,
        compiler_params=pltpu.CompilerParams(
            use_tc_tiling_on_sc=False, needs_layout_passes=False
        ),
        scratch_types=[
            pltpu.VMEM_SHARED((N_NODES, D_EDGE), jnp.float32),
            pltpu.VMEM((2, 2, TCOLS, 8, LANES), jnp.float32),
            pltpu.VMEM((2, TCOLS, LANES, D_EDGE), jnp.float32),
            pltpu.VMEM((4, SUB, SCAT), jnp.int32),
            pltpu.VMEM((2, SCAT * D_EDGE), jnp.float32),
            pltpu.SemaphoreType.DMA,
            pltpu.SemaphoreType.DMA,
            pltpu.SemaphoreType.DMA,
            pltpu.SemaphoreType.DMA,
        ],
    )
    def sc_kernel(e_hbm, i_hbm, out_hbm, acc, etbuf, ebuf, ibuf, obuf,
                  lsem0, lsem1, ssem0, ssem1):
        cid = lax.axis_index("c")
        sid = lax.axis_index("s")
        wid = sid * NUM_CORES + cid
        lsem = (lsem0, lsem1)
        ssem = (ssem0, ssem1)

        zstage = ebuf.at[0, 0]

        @pl.loop(0, LANES)
        def _(i):
            zstage[i, :] = jnp.zeros((D_EDGE,), jnp.float32)

        @pl.loop(0, ROWS_PER_SUBCORE // LANES)
        def _(k):
            pltpu.sync_copy(
                zstage, acc.at[pl.ds(sid * ROWS_PER_SUBCORE + k * LANES, LANES)]
            )

        _tail_base = sid * ROWS_PER_SUBCORE + (ROWS_PER_SUBCORE // LANES) * LANES
        _tail = ROWS_PER_SUBCORE % LANES
        pltpu.sync_copy(zstage.at[pl.ds(0, _tail)], acc.at[pl.ds(_tail_base, _tail)])

        plsc.subcore_barrier()

        lane = lax.iota(jnp.int32, 16)
        perms = [jnp.bitwise_and(lane + k, 15) for k in range(16)]
        fhi = lax.shift_right_logical(lane, 3)
        flo = jnp.bitwise_and(lane, 7)

        def start_load(slot, rd):
            c = wid + NUM_TILES * rd

            @pl.when(c < N_CHUNKS)
            def _():
                pltpu.async_copy(
                    e_hbm.at[:, pl.ds(c * TCOLS, TCOLS)], etbuf.at[slot],
                    lsem[slot])
                pltpu.async_copy(
                    i_hbm.at[pl.ds(c * SUB, SUB)], ibuf.at[rd % 4], lsem[slot])

        def wait_load(slot):
            pltpu.make_async_copy(
                e_hbm.at[:, pl.ds(0, TCOLS)], etbuf.at[slot], lsem[slot]).wait()
            pltpu.make_async_copy(
                i_hbm.at[pl.ds(0, SUB)], ibuf.at[0], lsem[slot]).wait()

        def wait_scatter(slot):
            for j in range(SUB):
                pltpu.make_async_copy(
                    out_hbm.at[pl.ds(0, SCAT * D_EDGE)], obuf.at[0],
                    ssem[slot]).wait()

        start_load(0, 0)
        start_load(1, 1)

        @pl.loop(0, SUPER)
        def _(r):
            for slot in range(2):
                rd = 2 * r + slot
                c = wid + NUM_TILES * rd

                @pl.when(c < N_CHUNKS)
                def _():
                    wait_load(slot)

                    @pl.when(rd >= 2)
                    def _():
                        wait_scatter(slot)

                    for jj in range(TCOLS):
                        jjv = jnp.full((16,), jj, jnp.int32)

                        @pl.loop(0, LANES, step=16)
                        def _(l0):
                            diag = [
                                jnp.full((16,), l0, jnp.int32) + perms[k]
                                for k in range(16)
                            ]
                            vals = [
                                plsc.load_gather(
                                    etbuf.at[slot], [fhi, jjv, flo, diag[k]])
                                for k in range(16)
                            ]
                            for k in range(16):
                                plsc.store_scatter(
                                    ebuf.at[slot],
                                    [jjv, diag[k], lane], vals[k])

                        pltpu.async_copy(
                            ebuf.at[slot, jj],
                            acc.at[ibuf.at[rd % 4, jj]],
                            ssem[slot],
                            add=True,
                        )
                    start_load(slot, rd + 2)

        wait_scatter(0)
        wait_scatter(1)

        plsc.subcore_barrier()

        OFULL = ROWS_PER_SUBCORE // SCAT
        OTAIL = ROWS_PER_SUBCORE % SCAT
        obase = (cid * N_NODES + sid * ROWS_PER_SUBCORE) * D_EDGE

        def relabel(oslot, nrows):
            t2 = ebuf.at[oslot, 0]

            @pl.loop(0, nrows)
            def _(i):
                obuf[oslot, pl.ds(i * D_EDGE, D_EDGE)] = t2[i, :]

        def owait(oslot, nrows):
            pltpu.make_async_copy(
                obuf.at[oslot, pl.ds(0, nrows * D_EDGE)],
                out_hbm.at[pl.ds(0, nrows * D_EDGE)], ssem[oslot]).wait()

        @pl.loop(0, OFULL // 2)
        def _(r):
            for oslot in range(2):
                k2 = 2 * r + oslot
                pltpu.sync_copy(
                    acc.at[pl.ds(sid * ROWS_PER_SUBCORE + k2 * SCAT, SCAT)],
                    ebuf.at[oslot, 0])

                @pl.when(r >= 1)
                def _():
                    owait(oslot, SCAT)

                relabel(oslot, SCAT)
                pltpu.async_copy(
                    obuf.at[oslot],
                    out_hbm.at[pl.ds(obase + k2 * SCAT * D_EDGE,
                                     SCAT * D_EDGE)],
                    ssem[oslot])

        pltpu.sync_copy(
            acc.at[pl.ds(sid * ROWS_PER_SUBCORE + OFULL * SCAT, OTAIL)],
            ebuf.at[0, 0, pl.ds(0, OTAIL)])
        owait(0, SCAT)
        relabel(0, OTAIL)
        pltpu.async_copy(
            obuf.at[0, pl.ds(0, OTAIL * D_EDGE)],
            out_hbm.at[pl.ds(obase + OFULL * SCAT * D_EDGE, OTAIL * D_EDGE)],
            ssem0)
        owait(1, SCAT)
        owait(0, OTAIL)

    return sc_kernel(e4, idx2)


PROWS = BLK * D_EDGE // 128


def _tc_dense_kernel(n_ref, p_ref, g_ref, w_ref, b_ref, o_ref):
    x = n_ref[...]
    pv = p_ref[0] + p_ref[1]
    wn = w_ref[0:D_NODE, :]
    we = w_ref[D_NODE:D_NODE + D_EDGE, :]
    wg = w_ref[D_NODE + D_EDGE:, :]
    y = jnp.dot(x.astype(jnp.bfloat16), wn.astype(jnp.bfloat16),
                preferred_element_type=jnp.float32)
    zs = [
        jnp.dot(pv[:, 16 * j:16 * (j + 1)], we,
                precision=lax.Precision.DEFAULT)
        for j in range(8)
    ]
    y = y + jnp.stack(zs, axis=1).reshape(BLK, D_OUT)
    y = y + jnp.dot(g_ref[...], wg, precision=lax.Precision.DEFAULT)
    y = y + b_ref[...]
    o_ref[...] = jnp.maximum(y, 0.0)


def _tc_dense(nodes, partials, globals_, W, b2):
    return pl.pallas_call(
        _tc_dense_kernel,
        grid=(N_BLKS,),
        in_specs=[
            pl.BlockSpec((BLK, D_NODE), lambda i: (i, 0)),
            pl.BlockSpec((NUM_CORES, PROWS, 128), lambda i: (0, i, 0)),
            pl.BlockSpec((1, D_GLOBAL), lambda i: (0, 0)),
            pl.BlockSpec((D_NODE + D_EDGE + D_GLOBAL, D_OUT), lambda i: (0, 0)),
            pl.BlockSpec((1, D_OUT), lambda i: (0, 0)),
        ],
        out_specs=pl.BlockSpec((BLK, D_OUT), lambda i: (i, 0)),
        out_shape=jax.ShapeDtypeStruct((N_NODES, D_OUT), jnp.float32),
    )(nodes, partials, globals_, W, b2)


def kernel(nodes, edges, receivers, senders, globals_, W, b):
    del senders
    idx2 = receivers.astype(jnp.int32).reshape(N_TILE_COLS, LANES)
    e4 = edges.T.reshape(NUM_CORES, 8, N_TILE_COLS, LANES).transpose(0, 2, 1, 3)
    partials1d = _sc_segment_sum(e4, idx2)
    pv = partials1d.reshape(NUM_CORES, N_NODES * D_EDGE // 128, 128)
    return _tc_dense(nodes, pv, globals_, W, b.reshape(1, D_OUT))
